# baseline (speedup 1.0000x reference)
import functools

import jax
import jax.numpy as jnp
from jax import lax
from jax.experimental import pallas as pl
from jax.experimental.pallas import tpu as pltpu
from jax.experimental.pallas import tpu_sc as plsc

NC = 2
NS = 16
LANES = 16
CH = 128
KTOT = 320
K0 = 216
K1 = KTOT - K0
KMAX = max(K0, K1)
G = 8
NCHUNKS = NS * KTOT


def _sc_partials(n, half_rows, base_flat, data_half, row2, col2, adj2):
  mesh = plsc.VectorSubcoreMesh(core_axis_name="c", subcore_axis_name="s")
  nhalf = half_rows * n

  @functools.partial(
      pl.kernel,
      out_type=jax.ShapeDtypeStruct((NC, n), jnp.float32),
      mesh=mesh,
      scratch_types=[
          pltpu.VMEM((KMAX, CH), jnp.int32),
          pltpu.VMEM((KMAX, CH), jnp.int32),
          pltpu.VMEM((KMAX, CH), jnp.float32),
          pltpu.VMEM((KMAX, CH), jnp.float32),
          pltpu.VMEM((n,), jnp.float32),
          pltpu.VMEM_SHARED((n,), jnp.float32),
          pltpu.SemaphoreType.DMA,
          pltpu.SemaphoreType.DMA,
      ],
  )
  def sc_kernel(data_hbm, row_hbm, col_hbm, adj_hbm, out_hbm,
                rowv, offv, adjv, valv, zerov, acc, gsem, ssem):
    c = lax.axis_index("c")
    s = lax.axis_index("s")
    kc = jnp.where(c == 0, K0, K1)
    start = pl.multiple_of(jnp.where(c == 0, s * K0, NS * K0 + s * K1), 8)

    pltpu.sync_copy(row_hbm.at[pl.ds(start, KMAX)], rowv)
    pltpu.sync_copy(col_hbm.at[pl.ds(start, KMAX)], offv)
    pltpu.sync_copy(adj_hbm.at[pl.ds(start, KMAX)], adjv)

    @pl.when(s == 0)
    def _():
      def zero_body(i, carry):
        zerov[pl.ds(i * LANES, LANES)] = jnp.zeros((LANES,), jnp.float32)
        return carry
      lax.fori_loop(0, n // LANES, zero_body, 0)
      pltpu.sync_copy(zerov, acc)

    def flat_body(t, carry):
      for j in range(CH // LANES):
        sl = pl.ds(j * LANES, LANES)
        l = rowv[t, sl] * n + offv[t, sl] - base_flat
        inside = jnp.logical_and(l >= 0, l < nhalf)
        offv[t, sl] = jnp.where(inside, l, 0)
        adjv[t, sl] = jnp.where(inside, adjv[t, sl], 0.0)
      return carry
    lax.fori_loop(0, kc, flat_body, 0)

    plsc.subcore_barrier()

    def group_body(g, carry):
      base = g * G
      for j in range(G):
        pltpu.make_async_copy(
            data_hbm.at[offv.at[base + j]], valv.at[base + j], gsem).start()
      for j in range(G):
        pltpu.make_async_copy(
            data_hbm.at[offv.at[base + j]], valv.at[base + j], gsem).wait()
      for j in range(G):
        for i in range(CH // LANES):
          sl = pl.ds(i * LANES, LANES)
          valv[base + j, sl] = valv[base + j, sl] * adjv[base + j, sl]
      for j in range(G):
        pltpu.async_copy(
            valv.at[base + j], acc.at[rowv.at[base + j]], ssem, add=True)
      for j in range(G):
        pltpu.make_async_copy(
            valv.at[base + j], acc.at[rowv.at[base + j]], ssem).wait()
      return carry
    lax.fori_loop(0, kc // G, group_body, 0)

    plsc.subcore_barrier()

    @pl.when(s == 0)
    def _():
      pltpu.sync_copy(acc, out_hbm.at[c])

  return sc_kernel(data_half, row2, col2, adj2)


def _tc_finish(n, pa, pb):
  def tc_body(pa_ref, pb_ref, alpha_ref, araw_ref):
    a = (jnp.sum(pa_ref[...], axis=0, keepdims=True)
         + jnp.sum(pb_ref[...], axis=0, keepdims=True))
    araw_ref[...] = a
    m = jnp.max(a)
    e = jnp.exp(a - m)
    alpha_ref[...] = e / jnp.sum(e)

  alpha2, araw2 = pl.pallas_call(
      tc_body,
      out_shape=(
          jax.ShapeDtypeStruct((1, n), jnp.float32),
          jax.ShapeDtypeStruct((1, n), jnp.float32),
      ),
  )(pa, pb)
  return alpha2.reshape(n), araw2.reshape(n)


def kernel(data_input, edge_index, adj_values):
  n = data_input.shape[0]
  e = edge_index.shape[1]
  half = n // 2

  rows_pad = NCHUNKS - K1 + KMAX
  e_pad = rows_pad * CH
  pad = e_pad - e

  row = jnp.pad(edge_index[0], (0, pad))
  col = jnp.pad(edge_index[1], (0, pad))
  adj = jnp.pad(adj_values, (0, pad))

  row2 = row.reshape(rows_pad, CH)
  col2 = col.reshape(rows_pad, CH)
  adj2 = adj.reshape(rows_pad, CH)

  data_a = data_input[:half].reshape(-1)
  data_b = data_input[half:].reshape(-1)

  pa = _sc_partials(n, half, 0, data_a, row2, col2, adj2)
  pb = _sc_partials(n, n - half, half * n, data_b, row2, col2, adj2)
  return _tc_finish(n, pa, pb)


# speedup vs baseline: 7.5160x; 7.5160x over previous
import functools

import jax
import jax.numpy as jnp
from jax import lax
from jax.experimental import pallas as pl
from jax.experimental.pallas import tpu as pltpu
from jax.experimental.pallas import tpu_sc as plsc

NC = 2
NS = 16
LANES = 16
CH = 128
KTOT = 320
K0 = 224
K1 = KTOT - K0
KMAX = max(K0, K1)
G = 16
NCHUNKS = NS * KTOT


def _sc_partials(n, data_flat, row2, col2, adj2):
  mesh = plsc.VectorSubcoreMesh(core_axis_name="c", subcore_axis_name="s")

  @functools.partial(
      pl.kernel,
      out_type=jax.ShapeDtypeStruct((NC, n), jnp.float32),
      mesh=mesh,
      scratch_types=[
          pltpu.VMEM((KMAX, CH), jnp.int32),
          pltpu.VMEM((KMAX, CH), jnp.int32),
          pltpu.VMEM((KMAX, CH), jnp.float32),
          pltpu.VMEM((KMAX, CH), jnp.float32),
          pltpu.VMEM((n,), jnp.float32),
          pltpu.VMEM_SHARED((n,), jnp.float32),
          pltpu.SemaphoreType.DMA,
          pltpu.SemaphoreType.DMA,
      ],
  )
  def sc_kernel(data_hbm, row_hbm, col_hbm, adj_hbm, out_hbm,
                rowv, offv, adjv, valv, zerov, acc, gsem, ssem):
    c = lax.axis_index("c")
    s = lax.axis_index("s")
    kc = jnp.where(c == 0, K0, K1)
    start = pl.multiple_of(jnp.where(c == 0, s * K0, NS * K0 + s * K1), 8)

    pltpu.sync_copy(row_hbm.at[pl.ds(start, KMAX)], rowv)
    pltpu.sync_copy(col_hbm.at[pl.ds(start, KMAX)], offv)
    pltpu.sync_copy(adj_hbm.at[pl.ds(start, KMAX)], adjv)

    @pl.when(s == 0)
    def _():
      def zero_body(i, carry):
        zerov[pl.ds(i * LANES, LANES)] = jnp.zeros((LANES,), jnp.float32)
        return carry
      lax.fori_loop(0, n // LANES, zero_body, 0)
      pltpu.sync_copy(zerov, acc)

    def flat_body(t, carry):
      for j in range(CH // LANES):
        sl = pl.ds(j * LANES, LANES)
        offv[t, sl] = rowv[t, sl] * n + offv[t, sl]
      return carry
    lax.fori_loop(0, kc, flat_body, 0)

    plsc.subcore_barrier()

    def group_body(g, carry):
      base = g * G
      for j in range(G):
        pltpu.make_async_copy(
            data_hbm.at[offv.at[base + j]], valv.at[base + j], gsem).start()
      for j in range(G):
        pltpu.make_async_copy(
            data_hbm.at[offv.at[base + j]], valv.at[base + j], gsem).wait()
      for j in range(G):
        for i in range(CH // LANES):
          sl = pl.ds(i * LANES, LANES)
          valv[base + j, sl] = valv[base + j, sl] * adjv[base + j, sl]
      for j in range(G):
        pltpu.async_copy(
            valv.at[base + j], acc.at[rowv.at[base + j]], ssem, add=True)
      for j in range(G):
        pltpu.make_async_copy(
            valv.at[base + j], acc.at[rowv.at[base + j]], ssem).wait()
      return carry
    lax.fori_loop(0, kc // G, group_body, 0)

    plsc.subcore_barrier()

    @pl.when(s == 0)
    def _():
      pltpu.sync_copy(acc, out_hbm.at[c])

  return sc_kernel(data_flat, row2, col2, adj2)


def _tc_finish(n, partials):
  def tc_body(p_ref, alpha_ref, araw_ref):
    a = jnp.sum(p_ref[...], axis=0, keepdims=True)
    araw_ref[...] = a
    m = jnp.max(a)
    e = jnp.exp(a - m)
    alpha_ref[...] = e / jnp.sum(e)

  alpha2, araw2 = pl.pallas_call(
      tc_body,
      out_shape=(
          jax.ShapeDtypeStruct((1, n), jnp.float32),
          jax.ShapeDtypeStruct((1, n), jnp.float32),
      ),
  )(partials)
  return alpha2.reshape(n), araw2.reshape(n)


def kernel(data_input, edge_index, adj_values):
  n = data_input.shape[0]
  e = edge_index.shape[1]

  rows_pad = NCHUNKS - K1 + KMAX
  e_pad = rows_pad * CH
  pad = e_pad - e

  row = jnp.pad(edge_index[0], (0, pad))
  col = jnp.pad(edge_index[1], (0, pad))
  adj = jnp.pad(adj_values, (0, pad))

  row2 = row.reshape(rows_pad, CH)
  col2 = col.reshape(rows_pad, CH)
  adj2 = adj.reshape(rows_pad, CH)
  data_flat = data_input.reshape(-1)

  partials = _sc_partials(n, data_flat, row2, col2, adj2)
  return _tc_finish(n, partials)
